# Initial kernel scaffold; baseline (speedup 1.0000x reference)
import jax
import jax.numpy as jnp
from jax.experimental import pallas as pl


def kernel(x, edge_index, edge_weight, W1, b1, ln_g, ln_b, W2, b2):
    raise NotImplementedError("write your pallas kernel here")


# speedup vs baseline: 4.6560x; 4.6560x over previous
import jax
import jax.numpy as jnp
from jax import lax
from jax.experimental import pallas as pl
from jax.experimental.pallas import tpu as pltpu
from jax.experimental.pallas import tpu_sc as plsc

NC = 2
NS = 16
L = 16
CH = 128


def _rsqrt_newton(x):
    i = lax.bitcast_convert_type(x, jnp.int32)
    i = 0x5F3759DF - (i >> 1)
    y = lax.bitcast_convert_type(i, jnp.float32)
    for _ in range(3):
        y = y * (1.5 - 0.5 * x * y * y)
    return y


def _make_prep(n_pad, e_pad):
    slc = n_pad // NS
    ept = e_pad // NS
    epw = e_pad // (NC * NS)
    mesh = plsc.VectorSubcoreMesh(core_axis_name="c", subcore_axis_name="s")

    def body(srcp, dstp, ewp, coef_o, dinv2_o,
             deg_s, dinv_s, dst_c, ew_c, slice_v, dinv_full,
             srcv, dstv, ewv, coefv):
        c = lax.axis_index("c")
        s = lax.axis_index("s")
        wid = c * NS + s
        zeros16 = jnp.zeros((L,), jnp.float32)

        def zb(i, _):
            slice_v[pl.ds(i * L, L)] = zeros16
            return 0
        lax.fori_loop(0, slc // L, zb, 0)
        pltpu.sync_copy(slice_v, deg_s.at[pl.ds(s * slc, slc)])
        plsc.subcore_barrier()

        def deg_body(k, _):
            eb = s * ept + k * CH
            pltpu.sync_copy(dstp.at[pl.ds(eb, CH)], dst_c)
            pltpu.sync_copy(ewp.at[pl.ds(eb, CH)], ew_c)
            pltpu.sync_copy(ew_c, deg_s.at[dst_c], add=True)
            return 0
        lax.fori_loop(0, ept // CH, deg_body, 0)
        plsc.subcore_barrier()

        base = s * slc
        pltpu.sync_copy(deg_s.at[pl.ds(base, slc)], slice_v)

        def dinv_body(i, _):
            x = slice_v[pl.ds(i * L, L)] + 1.0
            slice_v[pl.ds(i * L, L)] = _rsqrt_newton(x)
            return 0
        lax.fori_loop(0, slc // L, dinv_body, 0)
        pltpu.sync_copy(slice_v, dinv_s.at[pl.ds(base, slc)])

        def sq_body(i, _):
            y = slice_v[pl.ds(i * L, L)]
            slice_v[pl.ds(i * L, L)] = y * y
            return 0
        lax.fori_loop(0, slc // L, sq_body, 0)

        @pl.when(c == 0)
        def _():
            pltpu.sync_copy(slice_v, dinv2_o.at[pl.ds(base, slc)])
        plsc.subcore_barrier()

        pltpu.sync_copy(dinv_s, dinv_full)
        eb = wid * epw
        pltpu.sync_copy(srcp.at[pl.ds(eb, epw)], srcv)
        pltpu.sync_copy(dstp.at[pl.ds(eb, epw)], dstv)
        pltpu.sync_copy(ewp.at[pl.ds(eb, epw)], ewv)

        def coef_body(g, _):
            sl = pl.ds(g * L, L)
            a = plsc.load_gather(dinv_full, [srcv[sl]])
            b = plsc.load_gather(dinv_full, [dstv[sl]])
            coefv[sl] = a * b * ewv[sl]
            return 0
        lax.fori_loop(0, epw // L, coef_body, 0)
        pltpu.sync_copy(coefv, coef_o.at[pl.ds(eb, epw)])

    return pl.kernel(
        body,
        out_type=(jax.ShapeDtypeStruct((e_pad,), jnp.float32),
                  jax.ShapeDtypeStruct((n_pad,), jnp.float32)),
        mesh=mesh,
        scratch_types=[
            pltpu.VMEM_SHARED((n_pad,), jnp.float32),
            pltpu.VMEM_SHARED((n_pad,), jnp.float32),
            pltpu.VMEM((CH,), jnp.int32),
            pltpu.VMEM((CH,), jnp.float32),
            pltpu.VMEM((slc,), jnp.float32),
            pltpu.VMEM((n_pad,), jnp.float32),
            pltpu.VMEM((epw,), jnp.int32),
            pltpu.VMEM((epw,), jnp.int32),
            pltpu.VMEM((epw,), jnp.float32),
            pltpu.VMEM((epw,), jnp.float32),
        ],
        compiler_params=pltpu.CompilerParams(needs_layout_passes=False),
    )


def _make_spmm(n, dh, e_pad):
    ept = e_pad // NS
    napad = -(-n // (NS * CH)) * (NS * CH)
    rpt = napad // NS
    mesh = plsc.VectorSubcoreMesh(core_axis_name="c", subcore_axis_name="s")

    def body(srcp, dstp, coef, h2d, out,
             acc_s, src_c, dst_c, rows, coefv, gsem):
        c = lax.axis_index("c")
        s = lax.axis_index("s")
        cn = c * n
        zeros16 = jnp.zeros((L,), jnp.float32)

        def zb(i, _):
            for j in range(dh // L):
                rows[i, pl.ds(j * L, L)] = zeros16
            return 0
        lax.fori_loop(0, CH, zb, 0)
        for j in range(rpt // CH):
            pltpu.sync_copy(rows, acc_s.at[pl.ds(s * rpt + j * CH, CH)])
        plsc.subcore_barrier()

        pltpu.sync_copy(coef.at[pl.ds(s * ept, ept)], coefv)

        def chunk(k, _):
            eb = s * ept + k * CH
            pltpu.sync_copy(srcp.at[pl.ds(eb, CH)], src_c)
            pltpu.sync_copy(dstp.at[pl.ds(eb, CH)], dst_c)
            for j in range(CH // L):
                sl = pl.ds(j * L, L)
                src_c[sl] = src_c[sl] + cn
            pltpu.async_copy(h2d.at[src_c], rows, gsem).wait()

            def scale(k2, _):
                idx = jnp.zeros((L,), jnp.int32) + (k * CH + k2)
                cc = plsc.load_gather(coefv, [idx])
                for j in range(dh // L):
                    sl2 = pl.ds(j * L, L)
                    rows[k2, sl2] = rows[k2, sl2] * cc
                return 0
            lax.fori_loop(0, CH, scale, 0)
            pltpu.sync_copy(rows, acc_s.at[dst_c], add=True)
            return 0
        lax.fori_loop(0, ept // CH, chunk, 0)
        plsc.subcore_barrier()

        base = s * rpt
        full = n // rpt
        rem = n - full * rpt

        @pl.when(s < full)
        def _():
            for j in range(rpt // CH):
                r0 = base + j * CH
                pltpu.sync_copy(acc_s.at[pl.ds(r0, CH)], rows)
                pltpu.sync_copy(rows, out.at[pl.ds(cn + r0, CH)])

        if rem > 0:
            @pl.when(s == full)
            def _():
                off = 0
                szs = [CH] * (rem // CH) + ([rem % CH] if rem % CH else [])
                for sz in szs:
                    pltpu.sync_copy(acc_s.at[pl.ds(base + off, sz)],
                                    rows.at[pl.ds(0, sz)])
                    pltpu.sync_copy(rows.at[pl.ds(0, sz)],
                                    out.at[pl.ds(cn + base + off, sz)])
                    off += sz

    return pl.kernel(
        body,
        out_type=jax.ShapeDtypeStruct((NC * n, dh), jnp.float32),
        mesh=mesh,
        scratch_types=[
            pltpu.VMEM_SHARED((napad, dh), jnp.float32),
            pltpu.VMEM((CH,), jnp.int32),
            pltpu.VMEM((CH,), jnp.int32),
            pltpu.VMEM((CH, dh), jnp.float32),
            pltpu.VMEM((ept,), jnp.float32),
            pltpu.SemaphoreType.DMA,
        ],
        compiler_params=pltpu.CompilerParams(needs_layout_passes=False),
    )


def _mm1_body(x_ref, w_ref, o_ref):
    o_ref[...] = lax.dot_general(x_ref[...], w_ref[...],
                                 (((1,), (1,)), ((), ())),
                                 preferred_element_type=jnp.float32)


def _mid_body(s0, s1, h0, h1, d, b1, g, bb, w2, o_ref):
    z0 = s0[...] + d[...] * h0[...]
    z1 = s1[...] + d[...] * h1[...]
    z = jnp.concatenate([z0, z1], axis=1) + b1[...]
    z = jnp.maximum(z, 0.0)
    mu = jnp.mean(z, axis=1, keepdims=True)
    zc = z - mu
    var = jnp.mean(zc * zc, axis=1, keepdims=True)
    y = zc * lax.rsqrt(var + 1e-5) * g[...] + bb[...]
    o_ref[...] = lax.dot_general(y, w2[...], (((1,), (1,)), ((), ())),
                                 preferred_element_type=jnp.float32)


def _fin_body(s0, s1, h0, h1, d, b2, o_ref):
    z0 = s0[...] + d[...] * h0[...]
    z1 = s1[...] + d[...] * h1[...]
    o_ref[...] = jnp.concatenate([z0, z1], axis=1) + b2[...]


def kernel(x, edge_index, edge_weight, W1, b1, ln_g, ln_b, W2, b2):
    n, d_in = x.shape
    d_hid = W1.shape[0]
    d_out = W2.shape[0]
    dh = d_hid // NC
    e = edge_index.shape[1]

    step = NS * CH
    e_pad = -(-e // step) * step
    step2 = NC * NS * CH
    e_pad = -(-e_pad // step2) * step2
    n_pad = -(-n // (NS * L)) * (NS * L)

    src = edge_index[0].astype(jnp.int32)
    dst = edge_index[1].astype(jnp.int32)
    pad = e_pad - e
    srcp = jnp.concatenate([src, jnp.zeros((pad,), jnp.int32)])
    dstp = jnp.concatenate([dst, jnp.zeros((pad,), jnp.int32)])
    ewp = jnp.concatenate([edge_weight.astype(jnp.float32),
                           jnp.zeros((pad,), jnp.float32)])

    coef, dinv2p = _make_prep(n_pad, e_pad)(srcp, dstp, ewp)
    dinv2 = dinv2p[:n].reshape(n, 1)

    rb = 400
    g = n // rb
    f32 = jnp.float32

    h1 = pl.pallas_call(
        _mm1_body,
        grid=(NC, g),
        in_specs=[
            pl.BlockSpec((rb, d_in), lambda c, i: (i, 0)),
            pl.BlockSpec((dh, d_in), lambda c, i: (c, 0)),
        ],
        out_specs=pl.BlockSpec((rb, dh), lambda c, i: (c * (n // rb) + i, 0)),
        out_shape=jax.ShapeDtypeStruct((NC * n, dh), f32),
    )(x, W1)

    spmm = _make_spmm(n, dh, e_pad)
    scat1 = spmm(srcp, dstp, coef, h1)

    h2 = pl.pallas_call(
        _mid_body,
        grid=(NC, g),
        in_specs=[
            pl.BlockSpec((rb, dh), lambda c, i: (i, 0)),
            pl.BlockSpec((rb, dh), lambda c, i: (n // rb + i, 0)),
            pl.BlockSpec((rb, dh), lambda c, i: (i, 0)),
            pl.BlockSpec((rb, dh), lambda c, i: (n // rb + i, 0)),
            pl.BlockSpec((rb, 1), lambda c, i: (i, 0)),
            pl.BlockSpec((1, d_hid), lambda c, i: (0, 0)),
            pl.BlockSpec((1, d_hid), lambda c, i: (0, 0)),
            pl.BlockSpec((1, d_hid), lambda c, i: (0, 0)),
            pl.BlockSpec((dh, d_hid), lambda c, i: (c, 0)),
        ],
        out_specs=pl.BlockSpec((rb, dh), lambda c, i: (c * (n // rb) + i, 0)),
        out_shape=jax.ShapeDtypeStruct((NC * n, dh), f32),
    )(scat1, scat1, h1, h1, dinv2, b1.reshape(1, -1), ln_g.reshape(1, -1),
      ln_b.reshape(1, -1), W2)

    scat2 = spmm(srcp, dstp, coef, h2)

    out = pl.pallas_call(
        _fin_body,
        grid=(g,),
        in_specs=[
            pl.BlockSpec((rb, dh), lambda i: (i, 0)),
            pl.BlockSpec((rb, dh), lambda i: (n // rb + i, 0)),
            pl.BlockSpec((rb, dh), lambda i: (i, 0)),
            pl.BlockSpec((rb, dh), lambda i: (n // rb + i, 0)),
            pl.BlockSpec((rb, 1), lambda i: (i, 0)),
            pl.BlockSpec((1, d_out), lambda i: (0, 0)),
        ],
        out_specs=pl.BlockSpec((rb, d_out), lambda i: (i, 0)),
        out_shape=jax.ShapeDtypeStruct((n, d_out), f32),
    )(scat2, scat2, h2, h2, dinv2, b2.reshape(1, -1))

    return out


# speedup vs baseline: 7.1014x; 1.5252x over previous
import jax
import jax.numpy as jnp
from jax import lax
from jax.experimental import pallas as pl
from jax.experimental.pallas import tpu as pltpu
from jax.experimental.pallas import tpu_sc as plsc

NC = 2
NS = 16
L = 16
CH = 128


def _make_hist(n_pad, e_pad):
    epw = e_pad // (NC * NS)
    ncw = epw // CH
    slc = n_pad // NS
    mesh = plsc.VectorSubcoreMesh(core_axis_name="c", subcore_axis_name="s")

    def body(dst2d, ewp, deg_o, deg_s, dstv, ewv, slice_v, ssem):
        c = lax.axis_index("c")
        s = lax.axis_index("s")
        wid = c * NS + s
        zeros16 = jnp.zeros((L,), jnp.float32)

        def zb(i, _):
            slice_v[pl.ds(i * L, L)] = zeros16
            return 0
        lax.fori_loop(0, slc // L, zb, 0)
        pltpu.sync_copy(slice_v, deg_s.at[pl.ds(s * slc, slc)])
        plsc.subcore_barrier()

        pltpu.sync_copy(dst2d.at[pl.ds(wid * ncw, ncw)], dstv)
        pltpu.sync_copy(ewp.at[pl.ds(wid * epw, epw)], ewv)
        for k0 in range(0, ncw, 20):
            descs = [
                pltpu.async_copy(ewv.at[pl.ds(k * CH, CH)],
                                 deg_s.at[dstv.at[k]], ssem, add=True)
                for k in range(k0, min(k0 + 20, ncw))
            ]
            for d in descs:
                d.wait()
        plsc.subcore_barrier()

        pltpu.sync_copy(deg_s.at[pl.ds(s * slc, slc)], slice_v)
        pltpu.sync_copy(slice_v, deg_o.at[pl.ds(c * n_pad + s * slc, slc)])

    return pl.kernel(
        body,
        out_type=jax.ShapeDtypeStruct((NC * n_pad,), jnp.float32),
        mesh=mesh,
        scratch_types=[
            pltpu.VMEM_SHARED((n_pad,), jnp.float32),
            pltpu.VMEM((ncw, CH), jnp.int32),
            pltpu.VMEM((epw,), jnp.float32),
            pltpu.VMEM((slc,), jnp.float32),
            pltpu.SemaphoreType.DMA,
        ],
        compiler_params=pltpu.CompilerParams(needs_layout_passes=False),
    )


def _make_spmm(n, dh, e_pad):
    ept = e_pad // NS
    nck = ept // CH
    napad = -(-n // (NS * CH)) * (NS * CH)
    rpt = napad // NS
    mesh = plsc.VectorSubcoreMesh(core_axis_name="c", subcore_axis_name="s")

    def body(src2, dstp, ewp, h2d, out,
             acc_s, srcv, dstc0, dstc1, ewc0, ewc1, rows0, rows1,
             gsem0, gsem1, ssem0, ssem1, dsem0, dsem1, esem0, esem1):
        c = lax.axis_index("c")
        s = lax.axis_index("s")
        cn = c * n
        zeros16 = jnp.zeros((L,), jnp.float32)
        rows = (rows0, rows1)
        dstc = (dstc0, dstc1)
        ewc = (ewc0, ewc1)
        gsem = (gsem0, gsem1)
        ssem = (ssem0, ssem1)
        dsem = (dsem0, dsem1)
        esem = (esem0, esem1)

        def zb(i, _):
            for j in range(dh // L):
                rows0[i, pl.ds(j * L, L)] = zeros16
            return 0
        lax.fori_loop(0, CH, zb, 0)
        for j in range(rpt // CH):
            pltpu.sync_copy(rows0, acc_s.at[pl.ds(s * rpt + j * CH, CH)])
        plsc.subcore_barrier()

        eb = c * e_pad + s * ept
        pltpu.sync_copy(src2.at[pl.ds(eb, ept)], srcv)
        et = s * ept

        dd = {0: pltpu.async_copy(dstp.at[pl.ds(et, CH)], dstc[0], dsem[0])}
        ed = {0: pltpu.async_copy(ewp.at[pl.ds(et, CH)], ewc[0], esem[0])}
        gd = {0: pltpu.async_copy(h2d.at[srcv.at[pl.ds(0, CH)]],
                                  rows[0], gsem[0])}
        sd = {}
        for g in range(nck):
            b = g % 2
            if g + 1 < nck:
                if g >= 1:
                    sd[g - 1].wait()
                o1 = et + (g + 1) * CH
                dd[g + 1] = pltpu.async_copy(dstp.at[pl.ds(o1, CH)],
                                             dstc[1 - b], dsem[1 - b])
                ed[g + 1] = pltpu.async_copy(ewp.at[pl.ds(o1, CH)],
                                             ewc[1 - b], esem[1 - b])
                gd[g + 1] = pltpu.async_copy(
                    h2d.at[srcv.at[pl.ds((g + 1) * CH, CH)]],
                    rows[1 - b], gsem[1 - b])
            gd[g].wait()
            ed[g].wait()
            rb_ = rows[b]
            ew_ = ewc[b]

            def scale(k2, _, rb_=rb_, ew_=ew_):
                cc = plsc.load_gather(ew_, [jnp.zeros((L,), jnp.int32) + k2])
                for j in range(dh // L):
                    sl = pl.ds(j * L, L)
                    rb_[k2, sl] = rb_[k2, sl] * cc
                return 0
            lax.fori_loop(0, CH, scale, 0)
            dd[g].wait()
            sd[g] = pltpu.async_copy(rb_, acc_s.at[dstc[b]],
                                     ssem[b], add=True)
        sd[nck - 2].wait()
        sd[nck - 1].wait()
        plsc.subcore_barrier()

        base = s * rpt
        full = n // rpt
        rem = n - full * rpt

        @pl.when(s < full)
        def _():
            for j in range(rpt // CH):
                r0 = base + j * CH
                pltpu.sync_copy(acc_s.at[pl.ds(r0, CH)], rows0)
                pltpu.sync_copy(rows0, out.at[pl.ds(cn + r0, CH)])

        if rem > 0:
            @pl.when(s == full)
            def _():
                off = 0
                szs = [CH] * (rem // CH) + ([rem % CH] if rem % CH else [])
                for sz in szs:
                    pltpu.sync_copy(acc_s.at[pl.ds(base + off, sz)],
                                    rows0.at[pl.ds(0, sz)])
                    pltpu.sync_copy(rows0.at[pl.ds(0, sz)],
                                    out.at[pl.ds(cn + base + off, sz)])
                    off += sz

    return pl.kernel(
        body,
        out_type=jax.ShapeDtypeStruct((NC * n, dh), jnp.float32),
        mesh=mesh,
        scratch_types=[
            pltpu.VMEM_SHARED((napad, dh), jnp.float32),
            pltpu.VMEM((ept,), jnp.int32),
            pltpu.VMEM((CH,), jnp.int32),
            pltpu.VMEM((CH,), jnp.int32),
            pltpu.VMEM((CH,), jnp.float32),
            pltpu.VMEM((CH,), jnp.float32),
            pltpu.VMEM((CH, dh), jnp.float32),
            pltpu.VMEM((CH, dh), jnp.float32),
        ] + [pltpu.SemaphoreType.DMA] * 8,
        compiler_params=pltpu.CompilerParams(needs_layout_passes=False),
    )


def _mm1_body(x_ref, w_ref, da_ref, db_ref, o_ref):
    dinv = lax.rsqrt(da_ref[...] + db_ref[...] + 1.0)
    o_ref[...] = dinv * lax.dot_general(x_ref[...], w_ref[...],
                                        (((1,), (1,)), ((), ())),
                                        preferred_element_type=jnp.float32)


def _mid_body(s0, s1, h0, h1, da, db, b1, g, bb, w2, o_ref):
    dinv = lax.rsqrt(da[...] + db[...] + 1.0)
    z0 = dinv * (s0[...] + h0[...])
    z1 = dinv * (s1[...] + h1[...])
    z = jnp.concatenate([z0, z1], axis=1) + b1[...]
    z = jnp.maximum(z, 0.0)
    mu = jnp.mean(z, axis=1, keepdims=True)
    zc = z - mu
    var = jnp.mean(zc * zc, axis=1, keepdims=True)
    y = zc * lax.rsqrt(var + 1e-5) * g[...] + bb[...]
    o_ref[...] = dinv * lax.dot_general(y, w2[...], (((1,), (1,)), ((), ())),
                                        preferred_element_type=jnp.float32)


def _fin_body(s0, s1, h0, h1, da, db, b2, o_ref):
    dinv = lax.rsqrt(da[...] + db[...] + 1.0)
    z0 = dinv * (s0[...] + h0[...])
    z1 = dinv * (s1[...] + h1[...])
    o_ref[...] = jnp.concatenate([z0, z1], axis=1) + b2[...]


def kernel(x, edge_index, edge_weight, W1, b1, ln_g, ln_b, W2, b2):
    n, d_in = x.shape
    d_hid = W1.shape[0]
    d_out = W2.shape[0]
    dh = d_hid // NC
    e = edge_index.shape[1]

    step = NC * NS * CH
    e_pad = -(-e // step) * step
    n_pad = -(-n // (NS * L)) * (NS * L)

    src = edge_index[0].astype(jnp.int32)
    dst = edge_index[1].astype(jnp.int32)
    pad = e_pad - e
    srcp = jnp.concatenate([src, jnp.zeros((pad,), jnp.int32)])
    dstp = jnp.concatenate([dst, jnp.zeros((pad,), jnp.int32)])
    ewp = jnp.concatenate([edge_weight.astype(jnp.float32),
                           jnp.zeros((pad,), jnp.float32)])
    src2 = jnp.concatenate([srcp, srcp + n])
    dst2d = dstp.reshape(-1, CH)

    deg2 = _make_hist(n_pad, e_pad)(dst2d, ewp)
    dega = deg2[:n_pad].reshape(n_pad, 1)
    degb = deg2[n_pad:].reshape(n_pad, 1)

    rb = 400
    g = n // rb
    f32 = jnp.float32
    vspec = pl.BlockSpec((rb, 1), lambda c, i: (i, 0))
    bspec = pl.BlockSpec((1, d_hid), lambda c, i: (0, 0))

    h1 = pl.pallas_call(
        _mm1_body,
        grid=(NC, g),
        in_specs=[
            pl.BlockSpec((rb, d_in), lambda c, i: (i, 0)),
            pl.BlockSpec((dh, d_in), lambda c, i: (c, 0)),
            vspec, vspec,
        ],
        out_specs=pl.BlockSpec((rb, dh), lambda c, i: (c * (n // rb) + i, 0)),
        out_shape=jax.ShapeDtypeStruct((NC * n, dh), f32),
    )(x, W1, dega, degb)

    spmm = _make_spmm(n, dh, e_pad)
    scat1 = spmm(src2, dstp, ewp, h1)

    h2 = pl.pallas_call(
        _mid_body,
        grid=(NC, g),
        in_specs=[
            pl.BlockSpec((rb, dh), lambda c, i: (i, 0)),
            pl.BlockSpec((rb, dh), lambda c, i: (n // rb + i, 0)),
            pl.BlockSpec((rb, dh), lambda c, i: (i, 0)),
            pl.BlockSpec((rb, dh), lambda c, i: (n // rb + i, 0)),
            vspec, vspec, bspec, bspec, bspec,
            pl.BlockSpec((dh, d_hid), lambda c, i: (c, 0)),
        ],
        out_specs=pl.BlockSpec((rb, dh), lambda c, i: (c * (n // rb) + i, 0)),
        out_shape=jax.ShapeDtypeStruct((NC * n, dh), f32),
    )(scat1, scat1, h1, h1, dega, degb, b1.reshape(1, -1),
      ln_g.reshape(1, -1), ln_b.reshape(1, -1), W2)

    scat2 = spmm(src2, dstp, ewp, h2)

    out = pl.pallas_call(
        _fin_body,
        grid=(1, g),
        in_specs=[
            pl.BlockSpec((rb, dh), lambda c, i: (i, 0)),
            pl.BlockSpec((rb, dh), lambda c, i: (n // rb + i, 0)),
            pl.BlockSpec((rb, dh), lambda c, i: (i, 0)),
            pl.BlockSpec((rb, dh), lambda c, i: (n // rb + i, 0)),
            vspec, vspec,
            pl.BlockSpec((1, d_out), lambda c, i: (0, 0)),
        ],
        out_specs=pl.BlockSpec((rb, d_out), lambda c, i: (i, 0)),
        out_shape=jax.ShapeDtypeStruct((n, d_out), f32),
    )(scat2, scat2, h2, h2, dega, degb, b2.reshape(1, -1))

    return out
